# Initial kernel scaffold; baseline (speedup 1.0000x reference)
import jax
import jax.numpy as jnp
from jax.experimental import pallas as pl


def kernel(frame_position_ids, height_position_ids, width_position_ids, frame_position_encodings, height_position_encodings, width_position_encodings):
    raise NotImplementedError("write your pallas kernel here")


# speedup vs baseline: 2.3405x; 2.3405x over previous
import functools

import jax
import jax.numpy as jnp
from jax import lax
from jax.experimental import pallas as pl
from jax.experimental.pallas import tpu as pltpu
from jax.experimental.pallas import tpu_sc as plsc

B, S = 4, 8192
N = B * S
D1, D2, D3 = 682, 682, 684
D = D1 + D2 + D3
W1, W2 = 640, 640
W3 = D - W1 - W2
TW = 128
TB = D1 - TW
NC, NS = 2, 16
NW = NC * NS
CHUNK = 32
NCHUNKS = N // CHUNK
CPW = NCHUNKS // NW
L = 16


def _sc_gather(fid2, hid2, wid2, t1, t2, t3, tf, th):
    mesh = plsc.VectorSubcoreMesh(core_axis_name="c", subcore_axis_name="s")

    @functools.partial(
        pl.kernel,
        mesh=mesh,
        out_type=jax.ShapeDtypeStruct((N, D), jnp.float32),
        scratch_types=[
            pltpu.VMEM((CHUNK,), jnp.int32),
            pltpu.VMEM((CHUNK,), jnp.int32),
            pltpu.VMEM((CHUNK,), jnp.int32),
            pltpu.VMEM((CHUNK, D), jnp.float32),
            pltpu.VMEM((CHUNK, TW), jnp.float32),
            pltpu.VMEM((CHUNK, TW), jnp.float32),
            pltpu.SemaphoreType.DMA,
        ],
        compiler_params=pltpu.CompilerParams(needs_layout_passes=False),
    )
    def k(fid_hbm, hid_hbm, wid_hbm, t1_hbm, t2_hbm, t3_hbm, tf_hbm, th_hbm,
          out_hbm, fidx, hidx, widx, cbuf, tfbuf, thbuf, sem):
        w = lax.axis_index("s") * NC + lax.axis_index("c")

        def body(i, carry):
            j = pl.multiple_of(w * CPW + i, 1)
            pltpu.sync_copy(fid_hbm.at[j], fidx)
            pltpu.sync_copy(hid_hbm.at[j], hidx)
            pltpu.sync_copy(wid_hbm.at[j], widx)
            c1 = pltpu.async_copy(t1_hbm.at[fidx], cbuf.at[:, pl.ds(0, W1)], sem)
            c2 = pltpu.async_copy(t2_hbm.at[hidx], cbuf.at[:, pl.ds(W1, W2)], sem)
            c3 = pltpu.async_copy(t3_hbm.at[widx], cbuf.at[:, pl.ds(W1 + W2, W3)], sem)
            c4 = pltpu.async_copy(tf_hbm.at[fidx], tfbuf, sem)
            c5 = pltpu.async_copy(th_hbm.at[hidx], thbuf, sem)
            c1.wait()
            c2.wait()
            c3.wait()
            c4.wait()
            c5.wait()

            def fix(r, carry2):
                rv = jnp.full((L,), r, jnp.int32)
                for b in range(3):
                    cols = lax.iota(jnp.int32, L) + (80 + b * L)
                    v = plsc.load_gather(tfbuf, [rv, cols])
                    plsc.store_scatter(cbuf, [rv, cols + TB], v)
                for b in range(6):
                    cols = lax.iota(jnp.int32, L) + (32 + b * L)
                    v = plsc.load_gather(thbuf, [rv, cols])
                    plsc.store_scatter(cbuf, [rv, cols + (D1 + TB)], v)
                return carry2

            lax.fori_loop(0, CHUNK, fix, 0)
            pltpu.sync_copy(cbuf, out_hbm.at[pl.ds(j * CHUNK, CHUNK)])
            return carry

        lax.fori_loop(0, CPW, body, 0)

    return k(fid2, hid2, wid2, t1, t2, t3, tf, th)


def kernel(frame_position_ids, height_position_ids, width_position_ids,
           frame_position_encodings, height_position_encodings,
           width_position_encodings):
    fid2 = frame_position_ids.reshape(NCHUNKS, CHUNK).astype(jnp.int32)
    hid2 = height_position_ids.reshape(NCHUNKS, CHUNK).astype(jnp.int32)
    wid2 = width_position_ids.reshape(NCHUNKS, CHUNK).astype(jnp.int32)
    ftab = frame_position_encodings
    htab = height_position_encodings
    wtab = width_position_encodings
    t1 = ftab[:, :W1]
    t2 = jnp.pad(htab[:, : W2 - 42], ((0, 0), (42, 0)))
    t3 = jnp.pad(wtab, ((0, 0), (W3 - D3, 0)))
    tf = ftab[:, TB:]
    th = htab[:, TB:]
    out = _sc_gather(fid2, hid2, wid2, t1, t2, t3, tf, th)
    return out.reshape(B, S, D)


# speedup vs baseline: 2.9224x; 1.2486x over previous
import functools

import jax
import jax.numpy as jnp
from jax import lax
from jax.experimental import pallas as pl
from jax.experimental.pallas import tpu as pltpu
from jax.experimental.pallas import tpu_sc as plsc

B, S = 4, 8192
N = B * S
D1, D2, D3 = 682, 682, 684
D = D1 + D2 + D3
W1, W2 = 640, 640
W3 = D - W1 - W2
TW = 128
TB = D1 - TW
NC, NS = 2, 16
NW = NC * NS
CHUNK = 16
NCHUNKS = N // CHUNK
CPW = NCHUNKS // NW
L = 16


def _sc_gather(fid2, hid2, wid2, t1, t2, t3, tf, th):
    mesh = plsc.VectorSubcoreMesh(core_axis_name="c", subcore_axis_name="s")

    @functools.partial(
        pl.kernel,
        mesh=mesh,
        out_type=jax.ShapeDtypeStruct((N, D), jnp.float32),
        scratch_types=[
            pltpu.VMEM((CPW, CHUNK), jnp.int32),
            pltpu.VMEM((CPW, CHUNK), jnp.int32),
            pltpu.VMEM((CPW, CHUNK), jnp.int32),
            pltpu.VMEM((CHUNK, D), jnp.float32),
            pltpu.VMEM((CHUNK, D), jnp.float32),
            pltpu.VMEM((CHUNK, TW), jnp.float32),
            pltpu.VMEM((CHUNK, TW), jnp.float32),
            pltpu.VMEM((CHUNK, TW), jnp.float32),
            pltpu.VMEM((CHUNK, TW), jnp.float32),
            pltpu.SemaphoreType.DMA,
            pltpu.SemaphoreType.DMA,
            pltpu.SemaphoreType.DMA,
            pltpu.SemaphoreType.DMA,
        ],
        compiler_params=pltpu.CompilerParams(needs_layout_passes=False),
    )
    def k(fid_hbm, hid_hbm, wid_hbm, t1_hbm, t2_hbm, t3_hbm, tf_hbm, th_hbm,
          out_hbm, fidx, hidx, widx, cbuf0, cbuf1, tf0, tf1, th0, th1,
          sg0, sg1, so0, so1):
        w = lax.axis_index("s") * NC + lax.axis_index("c")
        base = w * CPW
        pltpu.sync_copy(fid_hbm.at[pl.ds(base, CPW)], fidx)
        pltpu.sync_copy(hid_hbm.at[pl.ds(base, CPW)], hidx)
        pltpu.sync_copy(wid_hbm.at[pl.ds(base, CPW)], widx)

        cbufs = (cbuf0, cbuf1)
        tfs = (tf0, tf1)
        ths = (th0, th1)
        sgs = (sg0, sg1)
        sos = (so0, so1)

        def gathers(i, slot, sem):
            pltpu.async_copy(t1_hbm.at[fidx.at[i]],
                             cbufs[slot].at[:, pl.ds(0, W1)], sem)
            pltpu.async_copy(t2_hbm.at[hidx.at[i]],
                             cbufs[slot].at[:, pl.ds(W1, W2)], sem)
            pltpu.async_copy(t3_hbm.at[widx.at[i]],
                             cbufs[slot].at[:, pl.ds(W1 + W2, W3)], sem)
            pltpu.async_copy(tf_hbm.at[fidx.at[i]], tfs[slot], sem)
            pltpu.async_copy(th_hbm.at[hidx.at[i]], ths[slot], sem)

        def wait_gathers(slot, sem):
            pltpu.make_async_copy(t1_hbm.at[fidx.at[0]],
                                  cbufs[slot].at[:, pl.ds(0, W1)], sem).wait()
            pltpu.make_async_copy(t2_hbm.at[hidx.at[0]],
                                  cbufs[slot].at[:, pl.ds(W1, W2)], sem).wait()
            pltpu.make_async_copy(t3_hbm.at[widx.at[0]],
                                  cbufs[slot].at[:, pl.ds(W1 + W2, W3)], sem).wait()
            pltpu.make_async_copy(tf_hbm.at[fidx.at[0]], tfs[slot], sem).wait()
            pltpu.make_async_copy(th_hbm.at[hidx.at[0]], ths[slot], sem).wait()

        def fixup(slot):
            cb, tfb, thb = cbufs[slot], tfs[slot], ths[slot]

            def fix(r, carry2):
                rv = jnp.full((L,), r, jnp.int32)
                for b in range(3):
                    cols = lax.iota(jnp.int32, L) + (80 + b * L)
                    v = plsc.load_gather(tfb, [rv, cols])
                    plsc.store_scatter(cb, [rv, cols + TB], v)
                for b in range(6):
                    cols = lax.iota(jnp.int32, L) + (32 + b * L)
                    v = plsc.load_gather(thb, [rv, cols])
                    plsc.store_scatter(cb, [rv, cols + (D1 + TB)], v)
                return carry2

            lax.fori_loop(0, CHUNK, fix, 0)

        def out_slice(i):
            return out_hbm.at[pl.ds((base + i) * CHUNK, CHUNK)]

        def issue_out(i, slot, sem):
            pltpu.async_copy(cbufs[slot], out_slice(i), sem)

        def wait_out(slot, sem):
            pltpu.make_async_copy(cbufs[slot], out_slice(0), sem).wait()

        gathers(0, 0, sg0)
        gathers(1, 1, sg1)

        def body(j, carry):
            c0 = pl.multiple_of(2 * j, 2)
            wait_gathers(0, sg0)
            fixup(0)
            issue_out(c0, 0, so0)
            wait_gathers(1, sg1)
            fixup(1)
            issue_out(c0 + 1, 1, so1)

            @pl.when(j + 1 < CPW // 2)
            def _():
                wait_out(0, so0)
                gathers(c0 + 2, 0, sg0)
                wait_out(1, so1)
                gathers(c0 + 3, 1, sg1)

            return carry

        lax.fori_loop(0, CPW // 2, body, 0)
        wait_out(0, so0)
        wait_out(1, so1)

    return k(fid2, hid2, wid2, t1, t2, t3, tf, th)


def kernel(frame_position_ids, height_position_ids, width_position_ids,
           frame_position_encodings, height_position_encodings,
           width_position_encodings):
    fid2 = frame_position_ids.reshape(NCHUNKS, CHUNK).astype(jnp.int32)
    hid2 = height_position_ids.reshape(NCHUNKS, CHUNK).astype(jnp.int32)
    wid2 = width_position_ids.reshape(NCHUNKS, CHUNK).astype(jnp.int32)
    ftab = frame_position_encodings
    htab = height_position_encodings
    wtab = width_position_encodings
    t1 = ftab[:, :W1]
    t2 = jnp.pad(htab[:, : W2 - 42], ((0, 0), (42, 0)))
    t3 = jnp.pad(wtab, ((0, 0), (W3 - D3, 0)))
    tf = ftab[:, TB:]
    th = htab[:, TB:]
    out = _sc_gather(fid2, hid2, wid2, t1, t2, t3, tf, th)
    return out.reshape(B, S, D)


# speedup vs baseline: 3.0254x; 1.0353x over previous
import functools

import jax
import jax.numpy as jnp
from jax import lax
from jax.experimental import pallas as pl
from jax.experimental.pallas import tpu as pltpu
from jax.experimental.pallas import tpu_sc as plsc

B, S = 4, 8192
N = B * S
D1, D2, D3 = 682, 682, 684
D = D1 + D2 + D3
W1, W2 = 640, 768
W3 = D - W1 - W2
TW = 128
TB = D1 - TW
NC, NS = 2, 16
NW = NC * NS
CHUNK = 16
NCHUNKS = N // CHUNK
CPW = NCHUNKS // NW
NBUF = 2
L = 16


def _sc_gather(fid2, hid2, wid2, t1, t2, t3, tf, tw):
    mesh = plsc.VectorSubcoreMesh(core_axis_name="c", subcore_axis_name="s")

    @functools.partial(
        pl.kernel,
        mesh=mesh,
        out_type=jax.ShapeDtypeStruct((N, D), jnp.float32),
        scratch_types=(
            [pltpu.VMEM((CPW, CHUNK), jnp.int32)] * 3
            + [pltpu.VMEM((CHUNK, D), jnp.float32)] * NBUF
            + [pltpu.VMEM((CHUNK, TW), jnp.float32)] * (2 * NBUF)
            + [pltpu.SemaphoreType.DMA] * (2 * NBUF)
        ),
        compiler_params=pltpu.CompilerParams(needs_layout_passes=False),
    )
    def k(fid_hbm, hid_hbm, wid_hbm, t1_hbm, t2_hbm, t3_hbm, tf_hbm, tw_hbm,
          out_hbm, fidx, hidx, widx,
          cbuf0, cbuf1, tf0, tf1, tw0, tw1,
          sg0, sg1, so0, so1):
        w = lax.axis_index("s") * NC + lax.axis_index("c")
        base = w * CPW
        pltpu.sync_copy(fid_hbm.at[pl.ds(base, CPW)], fidx)
        pltpu.sync_copy(hid_hbm.at[pl.ds(base, CPW)], hidx)
        pltpu.sync_copy(wid_hbm.at[pl.ds(base, CPW)], widx)

        cbufs = (cbuf0, cbuf1)
        tfs = (tf0, tf1)
        tws = (tw0, tw1)
        sgs = (sg0, sg1)
        sos = (so0, so1)

        def gathers(i, b):
            pltpu.async_copy(t1_hbm.at[fidx.at[i]],
                             cbufs[b].at[:, pl.ds(0, W1)], sgs[b])
            pltpu.async_copy(t2_hbm.at[hidx.at[i]],
                             cbufs[b].at[:, pl.ds(W1, W2)], sgs[b])
            pltpu.async_copy(t3_hbm.at[widx.at[i]],
                             cbufs[b].at[:, pl.ds(W1 + W2, W3)], sgs[b])
            pltpu.async_copy(tf_hbm.at[fidx.at[i]], tfs[b], sgs[b])
            pltpu.async_copy(tw_hbm.at[widx.at[i]], tws[b], sgs[b])

        def wait_gathers(b):
            pltpu.make_async_copy(t1_hbm.at[fidx.at[0]],
                                  cbufs[b].at[:, pl.ds(0, W1)], sgs[b]).wait()
            pltpu.make_async_copy(t2_hbm.at[hidx.at[0]],
                                  cbufs[b].at[:, pl.ds(W1, W2)], sgs[b]).wait()
            pltpu.make_async_copy(t3_hbm.at[widx.at[0]],
                                  cbufs[b].at[:, pl.ds(W1 + W2, W3)], sgs[b]).wait()
            pltpu.make_async_copy(tf_hbm.at[fidx.at[0]], tfs[b], sgs[b]).wait()
            pltpu.make_async_copy(tw_hbm.at[widx.at[0]], tws[b], sgs[b]).wait()

        def fixup(b):
            cb, tfb, twb = cbufs[b], tfs[b], tws[b]

            def fix(r, carry2):
                rv = jnp.full((L,), r, jnp.int32)
                for blk in range(3):
                    cols = lax.iota(jnp.int32, L) + (80 + blk * L)
                    v = plsc.load_gather(tfb, [rv, cols])
                    plsc.store_scatter(cb, [rv, cols + TB], v)
                for blk in range(3):
                    cols = lax.iota(jnp.int32, L) + (blk * L)
                    v = plsc.load_gather(twb, [rv, cols])
                    plsc.store_scatter(cb, [rv, cols + (D1 + D2)], v)
                return carry2

            lax.fori_loop(0, CHUNK, fix, 0)

        def issue_out(i, b):
            pltpu.async_copy(cbufs[b], out_hbm.at[pl.ds((base + i) * CHUNK, CHUNK)],
                             sos[b])

        def wait_out(b):
            pltpu.make_async_copy(cbufs[b], out_hbm.at[pl.ds(0, CHUNK)],
                                  sos[b]).wait()

        for b in range(NBUF):
            gathers(b, b)

        def body(j, carry):
            c0 = pl.multiple_of(NBUF * j, 1)
            for b in range(NBUF):
                wait_gathers(b)
                fixup(b)
                issue_out(c0 + b, b)

                @pl.when(c0 + b + NBUF < CPW)
                def _():
                    wait_out(b)
                    gathers(c0 + b + NBUF, b)

            return carry

        lax.fori_loop(0, CPW // NBUF, body, 0)
        for b in range(NBUF):
            wait_out(b)

    return k(fid2, hid2, wid2, t1, t2, t3, tf, tw)


def kernel(frame_position_ids, height_position_ids, width_position_ids,
           frame_position_encodings, height_position_encodings,
           width_position_encodings):
    fid2 = frame_position_ids.reshape(NCHUNKS, CHUNK).astype(jnp.int32)
    hid2 = height_position_ids.reshape(NCHUNKS, CHUNK).astype(jnp.int32)
    wid2 = width_position_ids.reshape(NCHUNKS, CHUNK).astype(jnp.int32)
    ftab = frame_position_encodings
    htab = height_position_encodings
    wtab = width_position_encodings
    t1 = ftab[:, :W1]
    t2 = jnp.pad(htab, ((0, 0), (42, 44)))
    t3 = wtab[:, D3 - W3:]
    tf = ftab[:, TB:]
    tw = wtab[:, :TW]
    out = _sc_gather(fid2, hid2, wid2, t1, t2, t3, tf, tw)
    return out.reshape(B, S, D)
